# Initial kernel scaffold; baseline (speedup 1.0000x reference)
import jax
import jax.numpy as jnp
from jax.experimental import pallas as pl


def kernel(x, pert_idx, G_coexpress, G_coexpress_weight, G_sim, G_sim_weight, params):
    raise NotImplementedError("write your pallas kernel here")


# speedup vs baseline: 13.7535x; 13.7535x over previous
import functools

import jax
import jax.numpy as jnp
from jax import lax
from jax.experimental import pallas as pl
from jax.experimental.pallas import tpu as pltpu
from jax.experimental.pallas import tpu_sc as plsc

G = 10000
P = 10000
H = 64
B = 16
NE = 160000
NW = 32
EPW = 5120
NEP = NW * EPW
CH = 128
NCH = EPW // CH

@functools.lru_cache(maxsize=None)
def _mesh():
    return plsc.VectorSubcoreMesh(core_axis_name="c", subcore_axis_name="s")


def _deg_body(cols_co, w_co, cols_sim, w_sim, zeros1, out_co, out_sim,
              idx_v, upd_v, sh_co, sh_sim):
    cid = lax.axis_index("c")
    sid = lax.axis_index("s")
    wid = sid * 2 + cid

    @pl.when(sid == 0)
    def _init():
        pltpu.sync_copy(zeros1, sh_co)
        pltpu.sync_copy(zeros1, sh_sim)

    plsc.subcore_barrier()

    def do_graph(cols_hbm, w_hbm, sh):
        def chunk(i, carry):
            base = wid * EPW + i * CH
            pltpu.sync_copy(cols_hbm.at[pl.ds(base, CH)], idx_v)
            pltpu.sync_copy(w_hbm.at[pl.ds(base, CH)], upd_v)
            pltpu.sync_copy(upd_v, sh.at[idx_v], add=True)
            return carry
        lax.fori_loop(0, NCH, chunk, 0)

    do_graph(cols_co, w_co, sh_co)
    do_graph(cols_sim, w_sim, sh_sim)
    plsc.subcore_barrier()

    @pl.when(sid == 0)
    def _out():
        pltpu.sync_copy(sh_co, out_co.at[cid])
        pltpu.sync_copy(sh_sim, out_sim.at[cid])


def _deg_call(*args):
    return functools.partial(
        pl.kernel,
        out_type=(jax.ShapeDtypeStruct((2, G, 1), jnp.float32),
                  jax.ShapeDtypeStruct((2, P, 1), jnp.float32)),
        mesh=_mesh(),
        scratch_types=[
            pltpu.VMEM((CH,), jnp.int32),
            pltpu.VMEM((CH, 1), jnp.float32),
            pltpu.VMEM_SHARED((G, 1), jnp.float32),
            pltpu.VMEM_SHARED((P, 1), jnp.float32),
        ],
    )(_deg_body)(*args)


def _lane_bcast(v16, k):
    dnums = lax.GatherDimensionNumbers(
        offset_dims=(), collapsed_slice_dims=(0,), start_index_map=(0,))
    idx = jnp.full((16, 1), k, jnp.int32)
    return lax.gather(v16, idx, dnums, (1,),
                      mode=lax.GatherScatterMode.PROMISE_IN_BOUNDS)


def _agg_body(rows_co, cols_co, w_co, xp_co, rows_sim, cols_sim, w_sim,
              xp_sim, zeros2, out_co, out_sim,
              idxr_v, idxc_v, w_v, rows_v, sh, sem):
    cid = lax.axis_index("c")
    sid = lax.axis_index("s")
    wid = sid * 2 + cid

    for rows_hbm, cols_hbm, w_hbm, xp_hbm, out in (
            (rows_co, cols_co, w_co, xp_co, out_co),
            (rows_sim, cols_sim, w_sim, xp_sim, out_sim)):

        @pl.when(sid == 0)
        def _init():
            pltpu.sync_copy(zeros2, sh)

        plsc.subcore_barrier()

        def chunk(i, carry):
            base = wid * EPW + i * CH
            pltpu.sync_copy(rows_hbm.at[pl.ds(base, CH)], idxr_v)
            pltpu.sync_copy(cols_hbm.at[pl.ds(base, CH)], idxc_v)
            pltpu.sync_copy(w_hbm.at[pl.ds(base, CH)], w_v)
            pltpu.async_copy(xp_hbm.at[idxr_v], rows_v, sem).wait()

            for j in range(CH // 16):
                wv16 = w_v[pl.ds(j * 16, 16)]
                for k in range(16):
                    wb = _lane_bcast(wv16, k)
                    e = j * 16 + k
                    for q in range(4):
                        rows_v[e, pl.ds(16 * q, 16)] = (
                            rows_v[e, pl.ds(16 * q, 16)] * wb)
            pltpu.sync_copy(rows_v, sh.at[idxc_v], add=True)
            return carry
        lax.fori_loop(0, NCH, chunk, 0)
        plsc.subcore_barrier()

        @pl.when(sid == 0)
        def _out():
            pltpu.sync_copy(sh, out.at[cid])


def _agg_call(*args):
    return functools.partial(
        pl.kernel,
        out_type=(jax.ShapeDtypeStruct((2, G, 2 * H), jnp.float32),
                  jax.ShapeDtypeStruct((2, P, 2 * H), jnp.float32)),
        mesh=_mesh(),
        scratch_types=[
            pltpu.VMEM((CH,), jnp.int32),
            pltpu.VMEM((CH,), jnp.int32),
            pltpu.VMEM((CH,), jnp.float32),
            pltpu.VMEM((CH, 2 * H), jnp.float32),
            pltpu.VMEM_SHARED((G, 2 * H), jnp.float32),
            pltpu.SemaphoreType.DMA,
        ],
    )(_agg_body)(*args)


_NBLK = 10
_BLK = G // _NBLK


def _prep_body(degp_co, degp_sim, emb_pos, pert_emb, gene_emb,
               xp_co_o, xp_sim_o, dinv_co_o, dinv_sim_o, xn_co_o, ge_rn_o):
    def renorm(t):
        n = jnp.sqrt(jnp.sum(t * t, axis=1, keepdims=True))
        return t * jnp.where(n > 1.0, 1.0 / (n + 1e-7), 1.0)

    dco = lax.rsqrt(1.0 + degp_co[0] + degp_co[1])
    dsi = lax.rsqrt(1.0 + degp_sim[0] + degp_sim[1])
    xn_co = renorm(emb_pos[...])
    xn_sim = renorm(pert_emb[...])
    z = jnp.zeros((_BLK, H), jnp.float32)
    xp_co_o[...] = jnp.concatenate([dco * xn_co, z], axis=1)
    xp_sim_o[...] = jnp.concatenate([dsi * xn_sim, z], axis=1)
    dinv_co_o[...] = dco
    dinv_sim_o[...] = dsi
    xn_co_o[...] = xn_co
    ge_rn_o[...] = renorm(gene_emb[...])


def _prep_call(degp_co, degp_sim, emb_pos, pert_emb, gene_emb):
    blk2 = lambda: pl.BlockSpec((_BLK, H), lambda i: (i, 0))
    return pl.pallas_call(
        _prep_body,
        grid=(_NBLK,),
        in_specs=[
            pl.BlockSpec((2, _BLK, 1), lambda i: (0, i, 0)),
            pl.BlockSpec((2, _BLK, 1), lambda i: (0, i, 0)),
            blk2(), blk2(), blk2(),
        ],
        out_specs=[
            pl.BlockSpec((_BLK, 2 * H), lambda i: (i, 0)),
            pl.BlockSpec((_BLK, 2 * H), lambda i: (i, 0)),
            pl.BlockSpec((_BLK, 1), lambda i: (i, 0)),
            pl.BlockSpec((_BLK, 1), lambda i: (i, 0)),
            blk2(), blk2(),
        ],
        out_shape=[
            jax.ShapeDtypeStruct((G, 2 * H), jnp.float32),
            jax.ShapeDtypeStruct((P, 2 * H), jnp.float32),
            jax.ShapeDtypeStruct((G, 1), jnp.float32),
            jax.ShapeDtypeStruct((P, 1), jnp.float32),
            jax.ShapeDtypeStruct((G, H), jnp.float32),
            jax.ShapeDtypeStruct((G, H), jnp.float32),
        ],
    )(degp_co, degp_sim, emb_pos, pert_emb, gene_emb)


def _etv2_body(aggp_co, xp_co, dinv_co, xn_co, ge_rn, bn_g, bn_b,
               sgp_WT, sgp_b,
               e2_W1T, e2_b1, e2_g1, e2_be1, e2_W2T, e2_b2,
               e_t0_o, e_c_o):
    f32 = jnp.float32
    dot = functools.partial(jnp.dot, preferred_element_type=f32)

    ge = ge_rn[...]
    m = jnp.mean(ge, axis=0, keepdims=True)
    v = jnp.mean((ge - m) ** 2, axis=0, keepdims=True)
    base = jax.nn.relu((ge - m) / jnp.sqrt(v + 1e-5) * bn_g[...] + bn_b[...])

    conv_co = dinv_co[...] * (aggp_co[0] + aggp_co[1] + xp_co[...])
    pos_t0 = dot(conv_co, sgp_WT[...]) + sgp_b[...]
    pos_c = dot(xn_co[...], sgp_WT[...]) + sgp_b[...]
    be_t0 = base + 0.2 * pos_t0
    be_c = base + 0.2 * pos_c

    h_t0 = dot(be_t0, e2_W1T[...]) + e2_b1[...]
    h_c = dot(be_c, e2_W1T[...]) + e2_b1[...]
    mh = (jnp.sum(h_t0, 0, keepdims=True)
          + 15.0 * jnp.sum(h_c, 0, keepdims=True)) / (16.0 * G)
    vh = (jnp.sum((h_t0 - mh) ** 2, 0, keepdims=True)
          + 15.0 * jnp.sum((h_c - mh) ** 2, 0, keepdims=True)) / (16.0 * G)
    sch = e2_g1[...] / jnp.sqrt(vh + 1e-5)

    def e2fin(h):
        hh = jax.nn.relu((h - mh) * sch + e2_be1[...])
        return jax.nn.relu(dot(hh, e2_W2T[...]) + e2_b2[...])
    e_t0_o[...] = e2fin(h_t0)
    e_c_o[...] = e2fin(h_c)


def _etv2_call(*args):
    return pl.pallas_call(
        _etv2_body,
        out_shape=[
            jax.ShapeDtypeStruct((G, H), jnp.float32),
            jax.ShapeDtypeStruct((G, H), jnp.float32),
        ],
    )(*args)


def _pert_body(aggp_sim, xp_sim, dinv_sim, e_t0, e_c, pi0, pi1,
               sgs_WT, sgs_b, pf_W1T, pf_b1, pf_g1, pf_be1, pf_W2T, pf_b2,
               pb_g, pb_b, T_t0_o, T_c_o, E2_o):
    f32 = jnp.float32
    dot = functools.partial(jnp.dot, preferred_element_type=f32)

    pre_sim = dinv_sim[...] * (aggp_sim[0] + aggp_sim[1] + xp_sim[...])
    iota = lax.broadcasted_iota(jnp.int32, (B, P), 1)
    onehot = ((pi0[...] == iota).astype(f32) + (pi1[...] == iota).astype(f32))
    pert_track = dot(dot(onehot, pre_sim), sgs_WT[...]) + 2.0 * sgs_b[...]
    hp = dot(pert_track, pf_W1T[...]) + pf_b1[...]
    mp = jnp.mean(hp, 0, keepdims=True)
    vp = jnp.mean((hp - mp) ** 2, 0, keepdims=True)
    hp = jax.nn.relu((hp - mp) / jnp.sqrt(vp + 1e-5) * pf_g1[...]
                     + pf_be1[...])
    emb_total = jax.nn.relu(dot(hp, pf_W2T[...]) + pf_b2[...])

    S_t0 = jnp.sum(e_t0[...], 0, keepdims=True)
    S_c = jnp.sum(e_c[...], 0, keepdims=True)
    mean_pb = (S_t0 + 15.0 * S_c
               + G * jnp.sum(emb_total, 0, keepdims=True)) / (16.0 * G)
    d = emb_total - mean_pb
    Q_t0 = jnp.sum(e_t0[...] ** 2, 0, keepdims=True)
    Q_c = jnp.sum(e_c[...] ** 2, 0, keepdims=True)
    d0 = d[0:1]
    var_pb = (Q_t0 + 2.0 * d0 * S_t0 + G * d0 ** 2 + 15.0 * Q_c
              + 2.0 * jnp.sum(d[1:], 0, keepdims=True) * S_c
              + G * jnp.sum(d[1:] ** 2, 0, keepdims=True)) / (16.0 * G)
    s_pb = pb_g[...] / jnp.sqrt(var_pb + 1e-5)
    T_t0_o[...] = e_t0[...] * s_pb
    T_c_o[...] = e_c[...] * s_pb
    E2_o[...] = d * s_pb + pb_b[...]


def _pert_call(*args):
    return pl.pallas_call(
        _pert_body,
        out_shape=[
            jax.ShapeDtypeStruct((G, H), jnp.float32),
            jax.ShapeDtypeStruct((G, H), jnp.float32),
            jax.ShapeDtypeStruct((B, H), jnp.float32),
        ],
    )(*args)


def _mom_body(Tcat, E2r, iw1, rw_W2, rw_b2c, ib1,
              Ssum_o, MM_o, Pm_o, cb_o):
    f32 = jnp.float32
    dot = functools.partial(jnp.dot, preferred_element_type=f32)
    b = pl.program_id(0)

    @pl.when(b == 0)
    def _init():
        Ssum_o[...] = jnp.zeros((1, H), f32)
        MM_o[...] = jnp.zeros((H, H), f32)
        Pm_o[...] = dot(iw1[...], rw_W2[...])
        cb_o[...] = dot(iw1[...], rw_b2c[...]) + ib1[...]

    Y = jax.nn.relu(Tcat[0] + E2r[0])
    Ssum_o[...] = Ssum_o[...] + jnp.sum(Y, 0, keepdims=True)
    MM_o[...] = MM_o[...] + lax.dot_general(
        Y, Y, (((0,), (0,)), ((), ())), preferred_element_type=f32)


def _t_spec():
    return pl.BlockSpec((1, G, H), lambda b: (jnp.minimum(b, 1), 0, 0))


def _mom_call(Tcat, E2r, iw1, rw_W2, rw_b2c, ib1):
    full = lambda s: pl.BlockSpec(s, lambda b: (0,) * len(s))
    return pl.pallas_call(
        _mom_body,
        grid=(B,),
        in_specs=[
            _t_spec(),
            pl.BlockSpec((1, 1, H), lambda b: (b, 0, 0)),
            full((G, H)), full((H, 2 * H)), full((H, 1)), full((G, 1)),
        ],
        out_specs=[full((1, H)), full((H, H)), full((G, 2 * H)),
                   full((G, 1))],
        out_shape=[
            jax.ShapeDtypeStruct((1, H), jnp.float32),
            jax.ShapeDtypeStruct((H, H), jnp.float32),
            jax.ShapeDtypeStruct((G, 2 * H), jnp.float32),
            jax.ShapeDtypeStruct((G, 1), jnp.float32),
        ],
    )(Tcat, E2r, iw1, rw_W2, rw_b2c, ib1)


def _wpass_body(Tcat, E2r, Ssum, MM, rw_W1T, rw_b1, rw_g1, rw_be1, Pm, cb,
                wcol_o):
    f32 = jnp.float32
    dot = functools.partial(jnp.dot, preferred_element_type=f32)

    m_y = Ssum[...] / (16.0 * G)
    M_y = MM[...] / (16.0 * G)
    mean0 = dot(m_y, rw_W1T[...])
    mean1 = mean0 + rw_b1[...]
    Eh2 = (jnp.sum(rw_W1T[...] * dot(M_y, rw_W1T[...]), 0, keepdims=True)
           + 2.0 * rw_b1[...] * mean0 + rw_b1[...] ** 2)
    var1 = Eh2 - mean1 ** 2
    sc1 = rw_g1[...] / jnp.sqrt(var1 + 1e-5)
    W1fT = rw_W1T[...] * sc1
    b1f = (rw_b1[...] - mean1) * sc1 + rw_be1[...]

    Y = jax.nn.relu(Tcat[0] + E2r[0])
    Hh = jax.nn.relu(dot(Y, W1fT) + b1f)
    wcol_o[0] = jnp.sum(Hh * Pm[...], 1, keepdims=True) + cb[...]


def _wpass_call(Tcat, E2r, Ssum, MM, rw_W1T, rw_b1, rw_g1, rw_be1, Pm, cb):
    full = lambda s: pl.BlockSpec(s, lambda b: (0,) * len(s))
    return pl.pallas_call(
        _wpass_body,
        grid=(B,),
        in_specs=[
            _t_spec(),
            pl.BlockSpec((1, 1, H), lambda b: (b, 0, 0)),
            full((1, H)), full((H, H)),
            full((H, 2 * H)), full((1, 2 * H)), full((1, 2 * H)),
            full((1, 2 * H)), full((G, 2 * H)), full((G, 1)),
        ],
        out_specs=pl.BlockSpec((1, G, 1), lambda b: (b, 0, 0)),
        out_shape=jax.ShapeDtypeStruct((B, G, 1), jnp.float32),
    )(Tcat, E2r, Ssum, MM, rw_W1T, rw_b1, rw_g1, rw_be1, Pm, cb)


def _fin_body(wcol, cg_W1, cg_b1c, cg_g1c, cg_be1c, cg_W2, cg_b2c,
              iw2_0, Cmat, ib2, xcol, out_o):
    f32 = jnp.float32
    dot = functools.partial(jnp.dot, preferred_element_type=f32)

    zc = dot(cg_W1[...], wcol[...]) + cg_b1c[...]
    mz = jnp.mean(zc, 1, keepdims=True)
    vz = jnp.mean((zc - mz) ** 2, 1, keepdims=True)
    zc = jax.nn.relu((zc - mz) / jnp.sqrt(vz + 1e-5) * cg_g1c[...]
                     + cg_be1c[...])
    cgec = dot(cg_W2[...], zc) + cg_b2c[...]

    out_o[...] = (wcol[...] * iw2_0[...] + dot(Cmat[...], cgec)
                  + ib2[...] + xcol[...])


def _fin_call(*args):
    return pl.pallas_call(
        _fin_body,
        out_shape=jax.ShapeDtypeStruct((G, B), jnp.float32),
    )(*args)


def kernel(x, pert_idx, G_coexpress, G_coexpress_weight, G_sim, G_sim_weight,
           params):
    p = params
    i32 = jnp.int32
    f32 = jnp.float32

    npad = NEP - NE
    spread = (jnp.arange(npad, dtype=i32) % G)
    def pad_edges(ei, w):
        rows = jnp.concatenate([ei[0].astype(i32), spread])
        cols = jnp.concatenate([ei[1].astype(i32), spread])
        wp = jnp.concatenate([w.astype(f32), jnp.zeros((npad,), f32)])
        return rows, cols, wp
    r_co, c_co, w_co = pad_edges(G_coexpress, G_coexpress_weight)
    r_si, c_si, w_si = pad_edges(G_sim, G_sim_weight)

    zeros1 = jnp.zeros((G, 1), f32)
    zeros2 = jnp.zeros((G, 2 * H), f32)

    degp_co, degp_sim = _deg_call(c_co, w_co.reshape(NEP, 1),
                                  c_si, w_si.reshape(NEP, 1), zeros1)

    xp_co_p, xp_sim_p, dinv_co, dinv_sim, xn_co, ge_rn = _prep_call(
        degp_co, degp_sim, p['emb_pos'], p['pert_emb'], p['gene_emb'])

    aggp_co_p, aggp_sim_p = _agg_call(r_co, c_co, w_co, xp_co_p,
                                      r_si, c_si, w_si, xp_sim_p, zeros2)
    aggp_co = aggp_co_p[:, :, :H]
    aggp_sim = aggp_sim_p[:, :, :H]
    xp_co = xp_co_p[:, :H]
    xp_sim = xp_sim_p[:, :H]

    row = lambda a: a.reshape(1, -1)
    col = lambda a: a.reshape(-1, 1)
    e_t0, e_c = _etv2_call(
        aggp_co, xp_co, dinv_co, xn_co, ge_rn,
        row(p['bn_emb_g']), row(p['bn_emb_b']),
        p['sg_pos_W'].T, row(p['sg_pos_b']),
        p['etv2_W1'].T, row(p['etv2_b1']), row(p['etv2_g1']),
        row(p['etv2_be1']), p['etv2_W2'].T, row(p['etv2_b2']))

    T_t0, T_c, E2 = _pert_call(
        aggp_sim, xp_sim, dinv_sim, e_t0, e_c,
        col(pert_idx[:, 0].astype(i32)), col(pert_idx[:, 1].astype(i32)),
        p['sg_sim_W'].T, row(p['sg_sim_b']),
        p['pf_W1'].T, row(p['pf_b1']), row(p['pf_g1']), row(p['pf_be1']),
        p['pf_W2'].T, row(p['pf_b2']),
        row(p['bn_pb_g']), row(p['bn_pb_b']))

    Tcat = jnp.stack([T_t0, T_c])
    E2r = E2.reshape(B, 1, H)

    Ssum, MM, Pm, cb = _mom_call(
        Tcat, E2r, p['indv_w1'][:, :, 0], p['rw_W2'], col(p['rw_b2']),
        p['indv_b1'])

    wcol3 = _wpass_call(
        Tcat, E2r, Ssum, MM,
        p['rw_W1'].T, row(p['rw_b1']), row(p['rw_g1']), row(p['rw_be1']),
        Pm, cb)

    outcol = _fin_call(
        wcol3.reshape(B, G).T,
        p['cg_W1'], col(p['cg_b1']), col(p['cg_g1']), col(p['cg_be1']),
        p['cg_W2'], col(p['cg_b2']),
        p['indv_w2'][0, :, 0:1], p['indv_w2'][0, :, 1:], p['indv_b2'].T,
        x.reshape(B, G).T)

    return outcol.T
